# baseline (speedup 1.0000x reference)
import functools

import jax
import jax.numpy as jnp
from jax import lax
from jax.experimental import pallas as pl
from jax.experimental.pallas import tpu as pltpu
from jax.experimental.pallas import tpu_sc as plsc

B = 16384
NUM = 13
F = 26
V = 100000
D = 32

NC = 2
NS = 16
NW = NC * NS

BF = B * F
PER_W = BF // NW
CHUNK = 13 * 128
NCHUNK = PER_W // CHUNK
KROWS = CHUNK // 128

assert PER_W * NW == BF and NCHUNK * CHUNK == PER_W


def _sc_gather(table_flat, idx):
    mesh = plsc.VectorSubcoreMesh(core_axis_name="c", subcore_axis_name="s")

    @functools.partial(
        pl.kernel,
        out_type=jax.ShapeDtypeStruct((NW * NCHUNK, CHUNK, D), jnp.float32),
        mesh=mesh,
        scratch_types=[
            pltpu.VMEM((KROWS, 128), jnp.int32),
            pltpu.VMEM((CHUNK, D), jnp.float32),
            pltpu.SemaphoreType.DMA,
        ],
        compiler_params=pltpu.CompilerParams(use_tc_tiling_on_sc=False),
    )
    def gather_kernel(table_hbm, idx_hbm, out_hbm, idx_v, rows_v, sem):
        wid = lax.axis_index("s") * NC + lax.axis_index("c")

        def body(s, _):
            pltpu.sync_copy(idx_hbm.at[wid, s], idx_v)
            copies = []
            for j in range(KROWS):
                copies.append(
                    pltpu.async_copy(
                        table_hbm.at[idx_v.at[j]],
                        rows_v.at[pl.ds(j * 128, 128)],
                        sem,
                    )
                )
            for c in copies:
                c.wait()
            pltpu.sync_copy(rows_v, out_hbm.at[wid * NCHUNK + s])
            return _

        lax.fori_loop(0, NCHUNK, body, None)

    return gather_kernel(table_flat, idx)


_BM = 1024
_INV_SQRT = float(1.0 / (1.0 + 1e-5) ** 0.5)


def _mlp_kernel(num_ref, emb_ref, w1n_ref, w1e_ref, b1_ref, g1_ref, be1_ref,
                w2_ref, b2_ref, g2_ref, be2_ref, wp_ref, bp_ref, out_ref):
    x = jnp.dot(num_ref[...], w1n_ref[...], preferred_element_type=jnp.float32)
    x = x + jnp.dot(emb_ref[...], w1e_ref[...], preferred_element_type=jnp.float32)
    x = (x + b1_ref[...]) * (g1_ref[...] * _INV_SQRT) + be1_ref[...]
    x = jnp.maximum(x, 0.0)
    x = jnp.dot(x, w2_ref[...], preferred_element_type=jnp.float32)
    x = (x + b2_ref[...]) * (g2_ref[...] * _INV_SQRT) + be2_ref[...]
    x = jnp.maximum(x, 0.0)
    x = jnp.dot(x, wp_ref[...], preferred_element_type=jnp.float32)
    out_ref[...] = x + bp_ref[...]


def _mlp(numerical, emb, W1, b1, g1, be1, W2, b2, g2, be2, Wp, bp):
    W1n = W1[:NUM]
    W1e = W1[NUM:]
    row = lambda v: v.reshape(1, -1)
    grid = (B // _BM,)
    full = lambda shape: pl.BlockSpec(shape, lambda i: (0, 0))
    return pl.pallas_call(
        _mlp_kernel,
        grid=grid,
        in_specs=[
            pl.BlockSpec((_BM, NUM), lambda i: (i, 0)),
            pl.BlockSpec((_BM, F * D), lambda i: (i, 0)),
            full((NUM, 512)),
            full((F * D, 512)),
            full((1, 512)), full((1, 512)), full((1, 512)),
            full((512, 256)),
            full((1, 256)), full((1, 256)), full((1, 256)),
            full((256, 768)),
            full((1, 768)),
        ],
        out_specs=pl.BlockSpec((_BM, 768), lambda i: (i, 0)),
        out_shape=jax.ShapeDtypeStruct((B, 768), jnp.float32),
    )(numerical, emb, W1n, W1e, row(b1), row(g1), row(be1),
      W2, row(b2), row(g2), row(be2), Wp, row(bp))


def kernel(numerical_data, categorical_data, emb_tables, W1, b1, g1, be1,
           W2, b2, g2, be2, Wp, bp):
    table_flat = emb_tables.reshape(F * V, D)
    idx = (categorical_data.astype(jnp.int32)
           + (jnp.arange(F, dtype=jnp.int32) * V)[None, :])
    idx = idx.reshape(NW, NCHUNK, KROWS, 128)
    emb = jnp.zeros((B, F * D), jnp.float32) + table_flat[0, 0] + idx[0, 0, 0, 0]
    return _mlp(numerical_data, emb, W1, b1, g1, be1, W2, b2, g2, be2, Wp, bp)


# speedup vs baseline: 5.6685x; 5.6685x over previous
import functools

import jax
import jax.numpy as jnp
from jax import lax
from jax.experimental import pallas as pl
from jax.experimental.pallas import tpu as pltpu
from jax.experimental.pallas import tpu_sc as plsc

B = 16384
NUM = 13
F = 26
V = 100000
D = 32

NC = 2
NS = 16
NW = NC * NS

GROUPS = [(4 * g, 4) for g in range(6)] + [(24, 2)]


def _sc_gather(table_flat, idx, nf):
    hits = B * nf
    per_w = hits // NW
    chunk = 1024
    krows = chunk // 128
    nchunk = per_w // chunk
    assert nchunk * chunk == per_w

    mesh = plsc.VectorSubcoreMesh(core_axis_name="c", subcore_axis_name="s")

    @functools.partial(
        pl.kernel,
        out_type=jax.ShapeDtypeStruct((NW * nchunk, chunk, D), jnp.float32),
        mesh=mesh,
        scratch_types=[
            pltpu.VMEM((krows, 128), jnp.int32),
            pltpu.VMEM((chunk, D), jnp.float32),
            pltpu.SemaphoreType.DMA,
        ],
        compiler_params=pltpu.CompilerParams(use_tc_tiling_on_sc=False),
    )
    def gather_kernel(table_hbm, idx_hbm, out_hbm, idx_v, rows_v, sem):
        wid = lax.axis_index("s") * NC + lax.axis_index("c")

        def body(s, _):
            pltpu.sync_copy(idx_hbm.at[wid, s], idx_v)
            copies = []
            for j in range(krows):
                copies.append(
                    pltpu.async_copy(
                        table_hbm.at[idx_v.at[j]],
                        rows_v.at[pl.ds(j * 128, 128)],
                        sem,
                    )
                )
            for cp in copies:
                cp.wait()
            pltpu.sync_copy(rows_v, out_hbm.at[wid * nchunk + s])
            return _

        lax.fori_loop(0, nchunk, body, None)

    return gather_kernel(table_flat, idx)


_BM = 1024
_INV_SQRT = float(1.0 / (1.0 + 1e-5) ** 0.5)


def _mlp_kernel(*refs):
    (num_ref, *emb_refs, w1n_ref, w1e0, w1e1, w1e2, w1e3, w1e4, w1e5, w1e6,
     b1_ref, g1_ref, be1_ref, w2_ref, b2_ref, g2_ref, be2_ref,
     wp_ref, bp_ref, out_ref) = refs
    w1es = (w1e0, w1e1, w1e2, w1e3, w1e4, w1e5, w1e6)
    x = jnp.dot(num_ref[...], w1n_ref[...], preferred_element_type=jnp.float32)
    for e_ref, w_ref in zip(emb_refs, w1es, strict=True):
        x = x + jnp.dot(e_ref[...], w_ref[...],
                        preferred_element_type=jnp.float32)
    x = (x + b1_ref[...]) * (g1_ref[...] * _INV_SQRT) + be1_ref[...]
    x = jnp.maximum(x, 0.0)
    x = jnp.dot(x, w2_ref[...], preferred_element_type=jnp.float32)
    x = (x + b2_ref[...]) * (g2_ref[...] * _INV_SQRT) + be2_ref[...]
    x = jnp.maximum(x, 0.0)
    x = jnp.dot(x, wp_ref[...], preferred_element_type=jnp.float32)
    out_ref[...] = x + bp_ref[...]


def _mlp(numerical, embs, W1, b1, g1, be1, W2, b2, g2, be2, Wp, bp):
    W1n = W1[:NUM]
    w1es = [W1[NUM + f0 * D: NUM + (f0 + nf) * D] for f0, nf in GROUPS]
    row = lambda v: v.reshape(1, -1)
    grid = (B // _BM,)
    full = lambda shape: pl.BlockSpec(shape, lambda i: (0, 0))
    emb_specs = [pl.BlockSpec((_BM, nf * D), lambda i: (i, 0))
                 for _, nf in GROUPS]
    w1e_specs = [full((nf * D, 512)) for _, nf in GROUPS]
    return pl.pallas_call(
        _mlp_kernel,
        grid=grid,
        in_specs=[
            pl.BlockSpec((_BM, NUM), lambda i: (i, 0)),
            *emb_specs,
            full((NUM, 512)),
            *w1e_specs,
            full((1, 512)), full((1, 512)), full((1, 512)),
            full((512, 256)),
            full((1, 256)), full((1, 256)), full((1, 256)),
            full((256, 768)),
            full((1, 768)),
        ],
        out_specs=pl.BlockSpec((_BM, 768), lambda i: (i, 0)),
        out_shape=jax.ShapeDtypeStruct((B, 768), jnp.float32),
    )(numerical, *embs, W1n, *w1es, row(b1), row(g1), row(be1),
      W2, row(b2), row(g2), row(be2), Wp, row(bp))


def kernel(numerical_data, categorical_data, emb_tables, W1, b1, g1, be1,
           W2, b2, g2, be2, Wp, bp):
    cat = categorical_data.astype(jnp.int32)
    embs = []
    for f0, nf in GROUPS:
        tab = emb_tables[f0:f0 + nf].reshape(nf * V, D)
        idx = cat[:, f0:f0 + nf] + (jnp.arange(nf, dtype=jnp.int32) * V)[None, :]
        nchunk = B * nf // NW // 1024
        idx = idx.reshape(NW, nchunk, 8, 128)
        rows = _sc_gather(tab, idx, nf)
        embs.append(rows.reshape(B, nf * D))
    return _mlp(numerical_data, embs, W1, b1, g1, be1,
                W2, b2, g2, be2, Wp, bp)


# speedup vs baseline: 25.7998x; 4.5515x over previous
import functools

import jax
import jax.numpy as jnp
from jax import lax
from jax.experimental import pallas as pl
from jax.experimental.pallas import tpu as pltpu
from jax.experimental.pallas import tpu_sc as plsc

B = 16384
NUM = 13
F = 26
V = 100000
D = 32

NC = 2
NS = 16
NW = NC * NS

BH = B // 2
NG = BH // 16


def _sc_gather_t(table_t, cat_t):
    mesh = plsc.VectorSubcoreMesh(core_axis_name="c", subcore_axis_name="s")

    @functools.partial(
        pl.kernel,
        out_type=jax.ShapeDtypeStruct((F * D, B), jnp.float32),
        mesh=mesh,
        scratch_types=[
            pltpu.VMEM((V,), jnp.float32),
            pltpu.VMEM((BH,), jnp.int32),
            pltpu.VMEM((BH,), jnp.float32),
            pltpu.SemaphoreType.DMA,
        ],
        compiler_params=pltpu.CompilerParams(
            use_tc_tiling_on_sc=True, needs_layout_passes=False),
    )
    def gather_kernel(tab_hbm, cat_hbm, out_hbm, row_v, vidx_v, stag_v, sem):
        wid = lax.axis_index("s") * NC + lax.axis_index("c")

        def field_body(f, _):
            r = f * D + wid
            pltpu.sync_copy(tab_hbm.at[r], row_v)
            for h in range(2):
                pltpu.sync_copy(cat_hbm.at[f, pl.ds(h * BH, BH)], vidx_v)

                def grp_body(g, _):
                    sl = pl.ds(g * 16, 16)
                    stag_v[sl] = plsc.load_gather(row_v, [vidx_v[sl]])
                    return _

                lax.fori_loop(0, NG, grp_body, None)
                pltpu.sync_copy(stag_v, out_hbm.at[r, pl.ds(h * BH, BH)])
            return _

        lax.fori_loop(0, F, field_body, None)

    return gather_kernel(table_t, cat_t)


_BM = 1024
_INV_SQRT = float(1.0 / (1.0 + 1e-5) ** 0.5)


def _mlp_kernel(num_ref, embt_ref, w1n_ref, w1e_ref, b1_ref, g1_ref, be1_ref,
                w2_ref, b2_ref, g2_ref, be2_ref, wp_ref, bp_ref, out_ref):
    x = jnp.dot(num_ref[...], w1n_ref[...], preferred_element_type=jnp.float32)
    x = x + lax.dot_general(embt_ref[...], w1e_ref[...],
                            (((0,), (0,)), ((), ())),
                            preferred_element_type=jnp.float32)
    x = (x + b1_ref[...]) * (g1_ref[...] * _INV_SQRT) + be1_ref[...]
    x = jnp.maximum(x, 0.0)
    x = jnp.dot(x, w2_ref[...], preferred_element_type=jnp.float32)
    x = (x + b2_ref[...]) * (g2_ref[...] * _INV_SQRT) + be2_ref[...]
    x = jnp.maximum(x, 0.0)
    x = jnp.dot(x, wp_ref[...], preferred_element_type=jnp.float32)
    out_ref[...] = x + bp_ref[...]


def _mlp(numerical, emb_t, W1, b1, g1, be1, W2, b2, g2, be2, Wp, bp):
    W1n = W1[:NUM]
    W1e = W1[NUM:]
    row = lambda v: v.reshape(1, -1)
    grid = (B // _BM,)
    full = lambda shape: pl.BlockSpec(shape, lambda i: (0, 0))
    return pl.pallas_call(
        _mlp_kernel,
        grid=grid,
        in_specs=[
            pl.BlockSpec((_BM, NUM), lambda i: (i, 0)),
            pl.BlockSpec((F * D, _BM), lambda i: (0, i)),
            full((NUM, 512)),
            full((F * D, 512)),
            full((1, 512)), full((1, 512)), full((1, 512)),
            full((512, 256)),
            full((1, 256)), full((1, 256)), full((1, 256)),
            full((256, 768)),
            full((1, 768)),
        ],
        out_specs=pl.BlockSpec((_BM, 768), lambda i: (i, 0)),
        out_shape=jax.ShapeDtypeStruct((B, 768), jnp.float32),
    )(numerical, emb_t, W1n, W1e, row(b1), row(g1), row(be1),
      W2, row(b2), row(g2), row(be2), Wp, row(bp))


def kernel(numerical_data, categorical_data, emb_tables, W1, b1, g1, be1,
           W2, b2, g2, be2, Wp, bp):
    table_t = emb_tables.transpose(0, 2, 1).reshape(F * D, V)
    cat_t = categorical_data.astype(jnp.int32).T
    emb_t = _sc_gather_t(table_t, cat_t)
    return _mlp(numerical_data, emb_t, W1, b1, g1, be1,
                W2, b2, g2, be2, Wp, bp)
